# baseline (speedup 1.0000x reference)
import functools

import jax
import jax.numpy as jnp
from jax.experimental import pallas as pl
from jax.experimental.pallas import tpu as pltpu

_LPAD = 128


def _fused_kernel(x_ref, w_ref, g_ref, b_ref, o_ref, xpad_ref, slab_ref, *,
                  NB, H, W, K, Cin, eps, neg_slope):
    HW = H * W
    pad = (K - 1) // 2
    col = jax.lax.broadcasted_iota(jnp.int32, (1, HW), 1) % W

    for s in range(NB):
        xpad_ref[:, :_LPAD] = jnp.zeros((Cin, _LPAD), xpad_ref.dtype)
        xpad_ref[:, _LPAD + HW:] = jnp.zeros((Cin, _LPAD), xpad_ref.dtype)
        xpad_ref[:, _LPAD:_LPAD + HW] = (
            x_ref[s].reshape(Cin, HW).astype(xpad_ref.dtype))

        for kh in range(K):
            for kw in range(K):
                t = kh * K + kw
                d = (kh - pad) * W + (kw - pad)
                sl = xpad_ref[:, _LPAD + d:_LPAD + d + HW]
                if kw < pad:
                    sl = jnp.where(col >= (pad - kw), sl, jnp.zeros_like(sl))
                elif kw > pad:
                    sl = jnp.where(col < W - (kw - pad), sl,
                                   jnp.zeros_like(sl))
                slab_ref[t * Cin:(t + 1) * Cin, :] = sl

        acc = jax.lax.dot_general(
            w_ref[...], slab_ref[...],
            dimension_numbers=(((1,), (0,)), ((), ())),
            preferred_element_type=jnp.float32)

        inv_n = 1.0 / HW
        mean = jnp.sum(acc, axis=1, keepdims=True) * inv_n
        ex2 = jnp.sum(acc * acc, axis=1, keepdims=True) * inv_n
        var = ex2 - mean * mean
        scale = g_ref[...] * jax.lax.rsqrt(var + eps)
        shift = b_ref[...] - mean * scale

        out = acc * scale + shift
        out = jnp.where(out >= 0, out, neg_slope * out)
        o_ref[s] = out


def kernel(x_nchw, weight_oikk, gamma, beta, *, eps=1e-5, neg_slope=0.01):
    N, Cin, H, W = x_nchw.shape
    Cout, Cin_w, K, K2 = weight_oikk.shape
    assert Cin == Cin_w and K == K2
    HW = H * W

    w2 = jnp.transpose(weight_oikk, (0, 2, 3, 1)).reshape(
        Cout, K * K * Cin).astype(jnp.bfloat16)
    g2 = gamma.reshape(Cout, 1).astype(jnp.float32)
    b2 = beta.reshape(Cout, 1).astype(jnp.float32)

    NB = 4
    body = functools.partial(_fused_kernel, NB=NB, H=H, W=W, K=K, Cin=Cin,
                             eps=eps, neg_slope=neg_slope)
    out = pl.pallas_call(
        body,
        out_shape=jax.ShapeDtypeStruct((N, Cout, HW), jnp.float32),
        grid=(N // NB,),
        in_specs=[
            pl.BlockSpec((NB, Cin, H, W), lambda n: (n, 0, 0, 0)),
            pl.BlockSpec((Cout, K * K * Cin), lambda n: (0, 0)),
            pl.BlockSpec((Cout, 1), lambda n: (0, 0)),
            pl.BlockSpec((Cout, 1), lambda n: (0, 0)),
        ],
        out_specs=pl.BlockSpec((NB, Cout, HW), lambda n: (n, 0, 0)),
        scratch_shapes=[
            pltpu.VMEM((Cin, _LPAD + HW + _LPAD), jnp.bfloat16),
            pltpu.VMEM((K * K * Cin, HW), jnp.bfloat16),
        ],
        compiler_params=pltpu.CompilerParams(
            dimension_semantics=("parallel",),
            vmem_limit_bytes=64 * 1024 * 1024),
    )(x_nchw, w2, g2, b2)
    return out.reshape(N, Cout, H, W)


# speedup vs baseline: 1.0589x; 1.0589x over previous
import functools

import jax
import jax.numpy as jnp
from jax.experimental import pallas as pl
from jax.experimental.pallas import tpu as pltpu

_LPAD = 128


def _fused_kernel(x_ref, w_ref, g_ref, b_ref, o_ref, xpad_ref, slab_ref, *,
                  NB, H, W, K, Cin, eps, neg_slope):
    HW = H * W
    pad = (K - 1) // 2
    col = jax.lax.broadcasted_iota(jnp.int32, (1, HW), 1) % W

    for s in range(NB):
        xpad_ref[:, :_LPAD] = jnp.zeros((Cin, _LPAD), xpad_ref.dtype)
        xpad_ref[:, _LPAD + HW:] = jnp.zeros((Cin, _LPAD), xpad_ref.dtype)
        xpad_ref[:, _LPAD:_LPAD + HW] = (
            x_ref[s].reshape(Cin, HW).astype(xpad_ref.dtype))

        for kh in range(K):
            for kw in range(K):
                t = kh * K + kw
                d = (kh - pad) * W + (kw - pad)
                sl = xpad_ref[:, _LPAD + d:_LPAD + d + HW]
                if kw < pad:
                    sl = jnp.where(col >= (pad - kw), sl, jnp.zeros_like(sl))
                elif kw > pad:
                    sl = jnp.where(col < W - (kw - pad), sl,
                                   jnp.zeros_like(sl))
                slab_ref[t * Cin:(t + 1) * Cin, :] = sl

        acc = jax.lax.dot_general(
            slab_ref[...], w_ref[...],
            dimension_numbers=(((0,), (0,)), ((), ())),
            preferred_element_type=jnp.float32)

        inv_n = 1.0 / HW
        mean = jnp.sum(acc, axis=0, keepdims=True) * inv_n
        ex2 = jnp.sum(acc * acc, axis=0, keepdims=True) * inv_n
        var = ex2 - mean * mean
        scale = g_ref[...] * jax.lax.rsqrt(var + eps)
        shift = b_ref[...] - mean * scale

        out = acc * scale + shift
        out = jnp.where(out >= 0, out, neg_slope * out)
        o_ref[s] = out


def kernel(x_nchw, weight_oikk, gamma, beta, *, eps=1e-5, neg_slope=0.01):
    N, Cin, H, W = x_nchw.shape
    Cout, Cin_w, K, K2 = weight_oikk.shape
    assert Cin == Cin_w and K == K2
    HW = H * W

    w2 = jnp.transpose(weight_oikk, (2, 3, 1, 0)).reshape(
        K * K * Cin, Cout).astype(jnp.bfloat16)
    g2 = gamma.reshape(1, Cout).astype(jnp.float32)
    b2 = beta.reshape(1, Cout).astype(jnp.float32)

    NB = 2
    body = functools.partial(_fused_kernel, NB=NB, H=H, W=W, K=K, Cin=Cin,
                             eps=eps, neg_slope=neg_slope)
    out = pl.pallas_call(
        body,
        out_shape=jax.ShapeDtypeStruct((N, HW, Cout), jnp.float32),
        grid=(N // NB,),
        in_specs=[
            pl.BlockSpec((NB, Cin, H, W), lambda n: (n, 0, 0, 0)),
            pl.BlockSpec((K * K * Cin, Cout), lambda n: (0, 0)),
            pl.BlockSpec((1, Cout), lambda n: (0, 0)),
            pl.BlockSpec((1, Cout), lambda n: (0, 0)),
        ],
        out_specs=pl.BlockSpec((NB, HW, Cout), lambda n: (n, 0, 0)),
        scratch_shapes=[
            pltpu.VMEM((Cin, _LPAD + HW + _LPAD), jnp.bfloat16),
            pltpu.VMEM((K * K * Cin, HW), jnp.bfloat16),
        ],
        compiler_params=pltpu.CompilerParams(
            dimension_semantics=("parallel",),
            vmem_limit_bytes=64 * 1024 * 1024),
    )(x_nchw, w2, g2, b2)
    return jnp.transpose(out, (0, 2, 1)).reshape(N, Cout, H, W)


# speedup vs baseline: 1.0871x; 1.0267x over previous
import functools

import jax
import jax.numpy as jnp
from jax.experimental import pallas as pl
from jax.experimental.pallas import tpu as pltpu

_LPAD = 128


def _fused_kernel(x_ref, w_ref, g_ref, b_ref, o_ref, xpad_ref, slab_ref, *,
                  NB, H, W, K, Cin, eps, neg_slope):
    HW = H * W
    pad = (K - 1) // 2
    col = jax.lax.broadcasted_iota(jnp.int32, (1, HW), 1) % W

    for s in range(NB):
        xpad_ref[:, :_LPAD] = jnp.zeros((Cin, _LPAD), xpad_ref.dtype)
        xpad_ref[:, _LPAD + HW:] = jnp.zeros((Cin, _LPAD), xpad_ref.dtype)
        xpad_ref[:, _LPAD:_LPAD + HW] = (
            x_ref[s].reshape(Cin, HW).astype(xpad_ref.dtype))

        for kh in range(K):
            for kw in range(K):
                t = kh * K + kw
                d = (kh - pad) * W + (kw - pad)
                sl = xpad_ref[:, _LPAD + d:_LPAD + d + HW]
                if kw < pad:
                    sl = jnp.where(col >= (pad - kw), sl, jnp.zeros_like(sl))
                elif kw > pad:
                    sl = jnp.where(col < W - (kw - pad), sl,
                                   jnp.zeros_like(sl))
                slab_ref[t * Cin:(t + 1) * Cin, :] = sl

        acc = jax.lax.dot_general(
            slab_ref[...], w_ref[...],
            dimension_numbers=(((0,), (0,)), ((), ())),
            preferred_element_type=jnp.float32)

        inv_n = 1.0 / HW
        mean = jnp.sum(acc, axis=0, keepdims=True) * inv_n
        ex2 = jnp.sum(acc * acc, axis=0, keepdims=True) * inv_n
        var = ex2 - mean * mean
        scale = g_ref[...] * jax.lax.rsqrt(var + eps)
        shift = b_ref[...] - mean * scale

        out = acc * scale + shift
        out = jnp.where(out >= 0, out, neg_slope * out)
        o_ref[s] = out


def kernel(x_nchw, weight_oikk, gamma, beta, *, eps=1e-5, neg_slope=0.01):
    N, Cin, H, W = x_nchw.shape
    Cout, Cin_w, K, K2 = weight_oikk.shape
    assert Cin == Cin_w and K == K2
    HW = H * W

    w2 = jnp.transpose(weight_oikk, (2, 3, 1, 0)).reshape(
        K * K * Cin, Cout).astype(jnp.bfloat16)
    g2 = gamma.reshape(1, Cout).astype(jnp.float32)
    b2 = beta.reshape(1, Cout).astype(jnp.float32)

    NB = 1
    body = functools.partial(_fused_kernel, NB=NB, H=H, W=W, K=K, Cin=Cin,
                             eps=eps, neg_slope=neg_slope)
    out = pl.pallas_call(
        body,
        out_shape=jax.ShapeDtypeStruct((N, HW, Cout), jnp.float32),
        grid=(N // NB,),
        in_specs=[
            pl.BlockSpec((NB, Cin, H, W), lambda n: (n, 0, 0, 0)),
            pl.BlockSpec((K * K * Cin, Cout), lambda n: (0, 0)),
            pl.BlockSpec((1, Cout), lambda n: (0, 0)),
            pl.BlockSpec((1, Cout), lambda n: (0, 0)),
        ],
        out_specs=pl.BlockSpec((NB, HW, Cout), lambda n: (n, 0, 0)),
        scratch_shapes=[
            pltpu.VMEM((Cin, _LPAD + HW + _LPAD), jnp.bfloat16),
            pltpu.VMEM((K * K * Cin, HW), jnp.bfloat16),
        ],
        compiler_params=pltpu.CompilerParams(
            dimension_semantics=("parallel",),
            vmem_limit_bytes=64 * 1024 * 1024),
    )(x_nchw, w2, g2, b2)
    return jnp.transpose(out, (0, 2, 1)).reshape(N, Cout, H, W)
